# baseline (speedup 1.0000x reference)
import functools

import jax
import jax.numpy as jnp
from jax import lax
from jax.experimental import pallas as pl
from jax.experimental.pallas import tpu as pltpu
from jax.experimental.pallas import tpu_sc as plsc

ROWS, COLS = 4096, 8192
L = 16
NC, NS = 2, 16
NW = NC * NS
RPW = ROWS // NW
RB = 4
NBLK = RPW // RB
SEG = COLS // L


def _body(x_hbm, out_hbm, buf):
    wid = lax.axis_index("s") * NC + lax.axis_index("c")
    elem_base = wid * (RPW * COLS)
    lane = lax.iota(jnp.int32, L)
    idx0 = lane * SEG

    def block(b, carry):
        e0 = elem_base + b * (RB * COLS)
        pltpu.sync_copy(x_hbm.at[pl.ds(e0, RB * COLS)], buf)

        def p0(j, c):
            idx = c[0]
            tots = []
            for r in range(RB):
                v = plsc.load_gather(buf, [idx + (r * COLS)])
                tots.append(c[1 + r] + v)
            return (idx + 1, *tots)

        z = jnp.zeros((L,), jnp.float32)
        c = lax.fori_loop(0, SEG, p0, (idx0, *([z] * RB)))
        tots = c[1:]

        accs = []
        for r in range(RB):
            t = tots[r]
            inc = lax.rev(plsc.cumsum(lax.rev(t, (0,))), (0,))
            accs.append(inc - t)

        def p1(j, c):
            idx = c[0]
            new = [idx - 1]
            for r in range(RB):
                ir = idx + (r * COLS)
                v = plsc.load_gather(buf, [ir])
                a = c[1 + r] + v
                plsc.store_scatter(buf, [ir], a)
                new.append(a)
            return tuple(new)

        lax.fori_loop(0, SEG, p1, (idx0 + (SEG - 1), *accs))

        pltpu.sync_copy(buf, out_hbm.at[pl.ds(e0, RB * COLS)])
        return carry

    lax.fori_loop(0, NBLK, block, 0)


def kernel(x):
    mesh = plsc.VectorSubcoreMesh(core_axis_name="c", subcore_axis_name="s")
    f = pl.kernel(
        _body,
        out_type=jax.ShapeDtypeStruct((ROWS * COLS,), jnp.float32),
        mesh=mesh,
        scratch_types=[pltpu.VMEM((RB * COLS,), jnp.float32)],
        compiler_params=pltpu.CompilerParams(needs_layout_passes=False),
    )
    return f(x.reshape(ROWS * COLS)).reshape(ROWS, COLS)


# speedup vs baseline: 1.0270x; 1.0270x over previous
import functools

import jax
import jax.numpy as jnp
from jax import lax
from jax.experimental import pallas as pl
from jax.experimental.pallas import tpu as pltpu
from jax.experimental.pallas import tpu_sc as plsc

ROWS, COLS = 4096, 8192
L = 16
NC, NS = 2, 16
NW = NC * NS
RPW = ROWS // NW
RB = 4
NBLK = RPW // RB
SEG = COLS // L


def _body(x_hbm, out_hbm, buf):
    wid = lax.axis_index("s") * NC + lax.axis_index("c")
    elem_base = wid * (RPW * COLS)
    lane = lax.iota(jnp.int32, L)
    idx0 = lane * SEG

    def block(b, carry):
        e0 = elem_base + b * (RB * COLS)
        pltpu.sync_copy(x_hbm.at[pl.ds(e0, RB * COLS)], buf)

        def p0(j, c):
            idx = c[0]
            tots = []
            for r in range(RB):
                v = plsc.load_gather(buf, [idx + (r * COLS)])
                tots.append(c[1 + r] + v)
            return (idx + 1, *tots)

        z = jnp.zeros((L,), jnp.float32)
        c = lax.fori_loop(0, SEG, p0, (idx0, *([z] * RB)), unroll=8)
        tots = c[1:]

        accs = []
        for r in range(RB):
            t = tots[r]
            inc = lax.rev(plsc.cumsum(lax.rev(t, (0,))), (0,))
            accs.append(inc - t)

        def p1(j, c):
            idx = c[0]
            new = [idx - 1]
            for r in range(RB):
                ir = idx + (r * COLS)
                v = plsc.load_gather(buf, [ir])
                a = c[1 + r] + v
                plsc.store_scatter(buf, [ir], a)
                new.append(a)
            return tuple(new)

        lax.fori_loop(0, SEG, p1, (idx0 + (SEG - 1), *accs), unroll=8)

        pltpu.sync_copy(buf, out_hbm.at[pl.ds(e0, RB * COLS)])
        return carry

    lax.fori_loop(0, NBLK, block, 0)


def kernel(x):
    mesh = plsc.VectorSubcoreMesh(core_axis_name="c", subcore_axis_name="s")
    f = pl.kernel(
        _body,
        out_type=jax.ShapeDtypeStruct((ROWS * COLS,), jnp.float32),
        mesh=mesh,
        scratch_types=[pltpu.VMEM((RB * COLS,), jnp.float32)],
        compiler_params=pltpu.CompilerParams(needs_layout_passes=False),
    )
    return f(x.reshape(ROWS * COLS)).reshape(ROWS, COLS)


# speedup vs baseline: 4.4534x; 4.3365x over previous
import functools

import jax
import jax.numpy as jnp
from jax import lax
from jax.experimental import pallas as pl
from jax.experimental.pallas import tpu as pltpu
from jax.experimental.pallas import tpu_sc as plsc

ROWS, COLS = 4096, 8192
L = 16
NC, NS = 2, 16
NW = NC * NS
RPW = ROWS // NW
RB = 4
NBLK = RPW // RB
VPR = COLS // L

_GDN = lax.GatherDimensionNumbers(
    offset_dims=(), collapsed_slice_dims=(0,), start_index_map=(0,))


def _bcast_last(s):
    last = jnp.full((L, 1), L - 1, jnp.int32)
    return lax.gather(s, last, _GDN, slice_sizes=(1,),
                      mode=lax.GatherScatterMode.PROMISE_IN_BOUNDS)


def _rc_row(buf, row_off):
    def step(k, u):
        base = row_off + (VPR - 1 - k) * L
        v = buf[pl.ds(base, L)]
        s = plsc.cumsum(v)
        r = _bcast_last(s)
        t = u + r
        buf[pl.ds(base, L)] = t - s + v
        return t

    lax.fori_loop(0, VPR, step, jnp.zeros((L,), jnp.float32), unroll=8)


def _body(x_hbm, out_hbm, buf):
    wid = lax.axis_index("s") * NC + lax.axis_index("c")
    elem_base = wid * (RPW * COLS)

    def block(b, carry):
        e0 = elem_base + b * (RB * COLS)
        pltpu.sync_copy(x_hbm.at[pl.ds(e0, RB * COLS)], buf)
        for r in range(RB):
            _rc_row(buf, r * COLS)
        pltpu.sync_copy(buf, out_hbm.at[pl.ds(e0, RB * COLS)])
        return carry

    lax.fori_loop(0, NBLK, block, 0)


def kernel(x):
    mesh = plsc.VectorSubcoreMesh(core_axis_name="c", subcore_axis_name="s")
    f = pl.kernel(
        _body,
        out_type=jax.ShapeDtypeStruct((ROWS * COLS,), jnp.float32),
        mesh=mesh,
        scratch_types=[pltpu.VMEM((RB * COLS,), jnp.float32)],
        compiler_params=pltpu.CompilerParams(needs_layout_passes=False),
    )
    return f(x.reshape(ROWS * COLS)).reshape(ROWS, COLS)
